# baseline (speedup 1.0000x reference)
import functools

import jax
import jax.numpy as jnp
from jax.experimental import pallas as pl
from jax.experimental.pallas import tpu as pltpu


def _dot_t(a, w, precision):
    return jax.lax.dot_general(
        a, w, (((1,), (1,)), ((), ())),
        preferred_element_type=jnp.float32, precision=precision)


def _nnue_kernel(wfts_ref, bfts_ref, ft_w_ref, stm_ref, ft_b_ref,
                 l1_w_ref, l1_b_ref, l2_w_ref, l2_b_ref, l3_w_ref, l3_b_ref,
                 out_ref, wp_acc, bp_acc,
                 *, num_k, tile_b, precision):
    k = pl.program_id(0)
    b = pl.program_id(1)
    rows = pl.ds(b * tile_b, tile_b)

    wp_part = _dot_t(wfts_ref[...], ft_w_ref[...], precision)
    bp_part = _dot_t(bfts_ref[...], ft_w_ref[...], precision)

    @pl.when(k == 0)
    def _init():
        wp_acc[rows, :] = wp_part
        bp_acc[rows, :] = bp_part

    @pl.when(k > 0)
    def _accum():
        wp_acc[rows, :] += wp_part
        bp_acc[rows, :] += bp_part

    @pl.when(k == num_k - 1)
    def _epilogue():
        ft_b = ft_b_ref[...]
        wp = wp_acc[rows, :] + ft_b
        bp = bp_acc[rows, :] + ft_b
        w, wpsqt = wp[:, :256], wp[:, 256:257]
        bb, bpsqt = bp[:, :256], bp[:, 256:257]
        s = stm_ref[...]
        acc = jnp.concatenate(
            [s * w + (1.0 - s) * bb, s * bb + (1.0 - s) * w], axis=1)
        x = jnp.clip(acc, 0.0, 1.0)
        x = jnp.clip(_dot_t(x, l1_w_ref[...], precision) + l1_b_ref[...], 0.0, 1.0)
        x = jnp.clip(_dot_t(x, l2_w_ref[...], precision) + l2_b_ref[...], 0.0, 1.0)
        x = _dot_t(x, l3_w_ref[...], precision)[:, :1] + l3_b_ref[0, 0]
        out_ref[...] = x + (wpsqt + bpsqt) * (s - 0.5)


@jax.jit
def kernel(wfts, bfts, stm, ft_W, ft_b, l1_W, l1_b, l2_W, l2_b, l3_W, l3_b):
    B, F = wfts.shape
    N = ft_W.shape[0]
    tile_b = min(512, B)
    tile_k = 5120 if F % 5120 == 0 else F
    num_b = B // tile_b
    num_k = F // tile_k
    precision = jax.lax.Precision.DEFAULT

    grid = (num_k, num_b)
    out = pl.pallas_call(
        functools.partial(_nnue_kernel, num_k=num_k, tile_b=tile_b,
                          precision=precision),
        grid=grid,
        in_specs=[
            pl.BlockSpec((tile_b, tile_k), lambda k, b: (b, k)),
            pl.BlockSpec((tile_b, tile_k), lambda k, b: (b, k)),
            pl.BlockSpec((N, tile_k), lambda k, b: (0, k)),
            pl.BlockSpec((tile_b, 1), lambda k, b: (b, 0)),
            pl.BlockSpec((1, N), lambda k, b: (0, 0)),
            pl.BlockSpec(l1_W.shape, lambda k, b: (0, 0)),
            pl.BlockSpec((1, l1_b.shape[0]), lambda k, b: (0, 0)),
            pl.BlockSpec(l2_W.shape, lambda k, b: (0, 0)),
            pl.BlockSpec((1, l2_b.shape[0]), lambda k, b: (0, 0)),
            pl.BlockSpec((128, l3_W.shape[1]), lambda k, b: (0, 0)),
            pl.BlockSpec(memory_space=pltpu.MemorySpace.SMEM),
        ],
        out_specs=pl.BlockSpec((tile_b, 1), lambda k, b: (b, 0)),
        out_shape=jax.ShapeDtypeStruct((B, 1), jnp.float32),
        scratch_shapes=[
            pltpu.VMEM((B, N), jnp.float32),
            pltpu.VMEM((B, N), jnp.float32),
        ],
        compiler_params=pltpu.CompilerParams(
            dimension_semantics=("arbitrary", "arbitrary")),
    )(wfts, bfts, ft_W, stm, ft_b.reshape(1, -1),
      l1_W, l1_b.reshape(1, -1), l2_W, l2_b.reshape(1, -1),
      jnp.pad(l3_W, ((0, 128 - l3_W.shape[0]), (0, 0))), l3_b.reshape(1, -1))
    return out


# speedup vs baseline: 1.0515x; 1.0515x over previous
import functools

import jax
import jax.numpy as jnp
from jax.experimental import pallas as pl
from jax.experimental.pallas import tpu as pltpu


def _dot_t(a, w, precision):
    return jax.lax.dot_general(
        a, w, (((1,), (1,)), ((), ())),
        preferred_element_type=jnp.float32, precision=precision)


def _nnue_kernel(wfts_ref, bfts_ref, ft_w_ref, stm_ref, ft_b_ref,
                 l1_w_ref, l1_b_ref, l2_w_ref, l2_b_ref, l3_w_ref, l3_b_ref,
                 out_ref, wp_acc, bp_acc,
                 *, num_k, tile_b, precision):
    k = pl.program_id(0)
    b = pl.program_id(1)
    rows = pl.ds(b * tile_b, tile_b)

    wp_part = _dot_t(wfts_ref[...], ft_w_ref[...], precision)
    bp_part = _dot_t(bfts_ref[...], ft_w_ref[...], precision)

    @pl.when(k == 0)
    def _init():
        wp_acc[rows, :] = wp_part
        bp_acc[rows, :] = bp_part

    @pl.when(k > 0)
    def _accum():
        wp_acc[rows, :] += wp_part
        bp_acc[rows, :] += bp_part

    @pl.when(k == num_k - 1)
    def _epilogue():
        ft_b = ft_b_ref[...]
        wp = wp_acc[rows, :] + ft_b
        bp = bp_acc[rows, :] + ft_b
        w, wpsqt = wp[:, :256], wp[:, 256:257]
        bb, bpsqt = bp[:, :256], bp[:, 256:257]
        s = stm_ref[...]
        acc = jnp.concatenate(
            [s * w + (1.0 - s) * bb, s * bb + (1.0 - s) * w], axis=1)
        x = jnp.clip(acc, 0.0, 1.0)
        x = jnp.clip(_dot_t(x, l1_w_ref[...], precision) + l1_b_ref[...], 0.0, 1.0)
        x = jnp.clip(_dot_t(x, l2_w_ref[...], precision) + l2_b_ref[...], 0.0, 1.0)
        x = _dot_t(x, l3_w_ref[...], precision)[:, :1] + l3_b_ref[0, 0]
        out_ref[...] = x + (wpsqt + bpsqt) * (s - 0.5)


@jax.jit
def kernel(wfts, bfts, stm, ft_W, ft_b, l1_W, l1_b, l2_W, l2_b, l3_W, l3_b):
    B, F = wfts.shape
    N = ft_W.shape[0]
    tile_b = min(1024, B)
    tile_k = 2560 if F % 2560 == 0 else F
    num_b = B // tile_b
    num_k = F // tile_k
    precision = jax.lax.Precision.DEFAULT

    grid = (num_k, num_b)
    out = pl.pallas_call(
        functools.partial(_nnue_kernel, num_k=num_k, tile_b=tile_b,
                          precision=precision),
        grid=grid,
        in_specs=[
            pl.BlockSpec((tile_b, tile_k), lambda k, b: (b, k)),
            pl.BlockSpec((tile_b, tile_k), lambda k, b: (b, k)),
            pl.BlockSpec((N, tile_k), lambda k, b: (0, k)),
            pl.BlockSpec((tile_b, 1), lambda k, b: (b, 0)),
            pl.BlockSpec((1, N), lambda k, b: (0, 0)),
            pl.BlockSpec(l1_W.shape, lambda k, b: (0, 0)),
            pl.BlockSpec((1, l1_b.shape[0]), lambda k, b: (0, 0)),
            pl.BlockSpec(l2_W.shape, lambda k, b: (0, 0)),
            pl.BlockSpec((1, l2_b.shape[0]), lambda k, b: (0, 0)),
            pl.BlockSpec((128, l3_W.shape[1]), lambda k, b: (0, 0)),
            pl.BlockSpec(memory_space=pltpu.MemorySpace.SMEM),
        ],
        out_specs=pl.BlockSpec((tile_b, 1), lambda k, b: (b, 0)),
        out_shape=jax.ShapeDtypeStruct((B, 1), jnp.float32),
        scratch_shapes=[
            pltpu.VMEM((B, N), jnp.float32),
            pltpu.VMEM((B, N), jnp.float32),
        ],
        compiler_params=pltpu.CompilerParams(
            dimension_semantics=("arbitrary", "arbitrary")),
    )(wfts, bfts, ft_W, stm, ft_b.reshape(1, -1),
      l1_W, l1_b.reshape(1, -1), l2_W, l2_b.reshape(1, -1),
      jnp.pad(l3_W, ((0, 128 - l3_W.shape[0]), (0, 0))), l3_b.reshape(1, -1))
    return out


# speedup vs baseline: 1.0957x; 1.0420x over previous
import functools

import jax
import jax.numpy as jnp
from jax.experimental import pallas as pl
from jax.experimental.pallas import tpu as pltpu

_CHUNK = 1024
_NBUF = 4


def _dot_t(a, w):
    return jax.lax.dot_general(
        a, w, (((1,), (1,)), ((), ())),
        preferred_element_type=jnp.float32,
        precision=jax.lax.Precision.DEFAULT)


def _nnue_kernel(wfts_hbm, bfts_hbm, ft_w_hbm, stm_ref, ft_b_ref,
                 l1_w_ref, l1_b_ref, l2_w_ref, l2_b_ref, l3_w_ref, l3_b_ref,
                 out_ref, wbuf, bbuf, fbuf, wacc, bacc, wsem, bsem, fsem,
                 *, num_k):
    def copies(k):
        slot = k % _NBUF
        cols = pl.ds(k * _CHUNK, _CHUNK)
        return (
            pltpu.make_async_copy(wfts_hbm.at[:, cols], wbuf.at[slot],
                                  wsem.at[slot]),
            pltpu.make_async_copy(bfts_hbm.at[:, cols], bbuf.at[slot],
                                  bsem.at[slot]),
            pltpu.make_async_copy(ft_w_hbm.at[:, cols], fbuf.at[slot],
                                  fsem.at[slot]),
        )

    for k in range(min(_NBUF, num_k)):
        for c in copies(k):
            c.start()

    for k in range(num_k):
        for c in copies(k):
            c.wait()
        slot = k % _NBUF
        wp_part = _dot_t(wbuf[slot], fbuf[slot])
        bp_part = _dot_t(bbuf[slot], fbuf[slot])
        if k == 0:
            wacc[...] = wp_part
            bacc[...] = bp_part
        else:
            wacc[...] += wp_part
            bacc[...] += bp_part
        if k + _NBUF < num_k:
            for c in copies(k + _NBUF):
                c.start()

    ft_b = ft_b_ref[...]
    wp = wacc[...] + ft_b
    bp = bacc[...] + ft_b
    w, wpsqt = wp[:, :256], wp[:, 256:257]
    bb, bpsqt = bp[:, :256], bp[:, 256:257]
    s = stm_ref[...]
    acc = jnp.concatenate(
        [s * w + (1.0 - s) * bb, s * bb + (1.0 - s) * w], axis=1)
    x = jnp.clip(acc, 0.0, 1.0)
    x = jnp.clip(_dot_t(x, l1_w_ref[...]) + l1_b_ref[...], 0.0, 1.0)
    x = jnp.clip(_dot_t(x, l2_w_ref[...]) + l2_b_ref[...], 0.0, 1.0)
    x = _dot_t(x, l3_w_ref[...])[:, :1] + l3_b_ref[0, 0]
    out_ref[...] = x + (wpsqt + bpsqt) * (s - 0.5)


@jax.jit
def kernel(wfts, bfts, stm, ft_W, ft_b, l1_W, l1_b, l2_W, l2_b, l3_W, l3_b):
    B, F = wfts.shape
    N = ft_W.shape[0]
    num_k = F // _CHUNK
    assert F % _CHUNK == 0

    any_spec = pl.BlockSpec(memory_space=pltpu.MemorySpace.HBM)
    vmem_spec = pl.BlockSpec(memory_space=pltpu.MemorySpace.VMEM)
    out = pl.pallas_call(
        functools.partial(_nnue_kernel, num_k=num_k),
        in_specs=[
            any_spec,
            any_spec,
            any_spec,
            vmem_spec,
            vmem_spec,
            vmem_spec,
            vmem_spec,
            vmem_spec,
            vmem_spec,
            vmem_spec,
            pl.BlockSpec(memory_space=pltpu.MemorySpace.SMEM),
        ],
        out_specs=vmem_spec,
        out_shape=jax.ShapeDtypeStruct((B, 1), jnp.float32),
        scratch_shapes=[
            pltpu.VMEM((_NBUF, B, _CHUNK), jnp.float32),
            pltpu.VMEM((_NBUF, B, _CHUNK), jnp.float32),
            pltpu.VMEM((_NBUF, N, _CHUNK), jnp.float32),
            pltpu.VMEM((B, N), jnp.float32),
            pltpu.VMEM((B, N), jnp.float32),
            pltpu.SemaphoreType.DMA((_NBUF,)),
            pltpu.SemaphoreType.DMA((_NBUF,)),
            pltpu.SemaphoreType.DMA((_NBUF,)),
        ],
    )(wfts, bfts, ft_W, stm, ft_b.reshape(1, -1),
      l1_W, l1_b.reshape(1, -1), l2_W, l2_b.reshape(1, -1),
      jnp.pad(l3_W, ((0, 128 - l3_W.shape[0]), (0, 0))), l3_b.reshape(1, -1))
    return out
